# baseline (speedup 1.0000x reference)
import functools

import jax
import jax.numpy as jnp
from jax import lax
from jax.experimental import pallas as pl
from jax.experimental.pallas import tpu as pltpu
from jax.experimental.pallas import tpu_sc as plsc

D_MODEL = 64
SCALE = 8.0

_NC = 2
_NS = 16
_NW = _NC * _NS
_G = 128
_NBUF = 4


@functools.lru_cache(maxsize=None)
def _make_kernel(n_groups):
    mesh = plsc.VectorSubcoreMesh(core_axis_name="c", subcore_axis_name="s")

    @functools.partial(
        pl.kernel,
        mesh=mesh,
        out_type=jax.ShapeDtypeStruct((_NW * n_groups, _G, D_MODEL), jnp.float32),
        scratch_types=(
            [pltpu.VMEM((n_groups, _G), jnp.int32)]
            + [pltpu.VMEM((_G, D_MODEL), jnp.float32) for _ in range(2 * _NBUF)]
            + [pltpu.SemaphoreType.DMA for _ in range(2 * _NBUF)]
        ),
        compiler_params=pltpu.CompilerParams(use_tc_tiling_on_sc=False),
    )
    def emb(x_hbm, table_hbm, out_hbm, idx_v, *rest):
        rows = rest[:_NBUF]
        obuf = rest[_NBUF:2 * _NBUF]
        gsem = rest[2 * _NBUF:3 * _NBUF]
        osem = rest[3 * _NBUF:4 * _NBUF]

        wid = lax.axis_index("s") * _NC + lax.axis_index("c")
        pltpu.sync_copy(x_hbm.at[wid], idx_v)

        def start_gather(g, b):
            pltpu.async_copy(table_hbm.at[idx_v.at[g]], rows[b], gsem[b])

        def wait_gather(g, b):
            pltpu.make_async_copy(table_hbm.at[idx_v.at[g]], rows[b], gsem[b]).wait()

        def start_out(g, b):
            pltpu.async_copy(obuf[b], out_hbm.at[wid * n_groups + g], osem[b])

        def wait_out(b):
            pltpu.make_async_copy(obuf[b], out_hbm.at[0], osem[b]).wait()

        def scale_rows(b):
            src = rows[b]
            dst = obuf[b]

            def mul_row(r, carry):
                for cc in range(D_MODEL // 16):
                    sl = pl.ds(cc * 16, 16)
                    dst[r, sl] = src[r, sl] * SCALE
                return carry

            lax.fori_loop(0, _G, mul_row, 0, unroll=4)

        for b in range(_NBUF):
            start_gather(b, b)

        for b in range(_NBUF):
            g = b
            wait_gather(g, b)
            scale_rows(b)
            start_gather(g + _NBUF, b)
            start_out(g, b)

        n_blocks = n_groups // _NBUF

        def block(blk, carry):
            for b in range(_NBUF):
                g = blk * _NBUF + b
                wait_gather(g, b)
                wait_out(b)
                scale_rows(b)
                ng = g + _NBUF

                @pl.when(ng < n_groups)
                def _():
                    start_gather(ng, b)

                start_out(g, b)
            return carry

        lax.fori_loop(1, n_blocks, block, 0)

        for b in range(_NBUF):
            wait_out(b)

    return emb


def kernel(x, table):
    bsz, seq = x.shape
    total = bsz * seq
    n_groups = total // (_NW * _G)
    x3 = x.astype(jnp.int32).reshape(_NW, n_groups, _G)
    out = _make_kernel(n_groups)(x3, table)
    return out.reshape(bsz, seq, D_MODEL)
